# baseline (speedup 1.0000x reference)
import jax
import jax.numpy as jnp
from jax.experimental import pallas as pl
from jax.experimental.pallas import tpu as pltpu

N = 100000
D = 128
G = 256
BLOCK = 10000
NB = N // BLOCK

NEG = -1e30


def _body(x_ref, w_ref, b_ref, q_ref, ones_ref, lo_ref, hi_ref,
          out_ref, m_ref, d_ref, s_ref):
    i = pl.program_id(0)

    @pl.when(i == 0)
    def _init():
        m_ref[...] = jnp.full((1, G), NEG, jnp.float32)
        d_ref[...] = jnp.zeros((1, G), jnp.float32)
        s_ref[...] = jnp.zeros((D, G), jnp.float32)

    xb = x_ref[...]
    g = jnp.tanh(
        jax.lax.dot_general(
            xb, w_ref[...], (((1,), (1,)), ((), ())),
            preferred_element_type=jnp.float32,
        )
        + b_ref[...]
    )
    score = jax.lax.dot_general(
        g, q_ref[...], (((1,), (0,)), ((), ())),
        preferred_element_type=jnp.float32,
    )

    row = jax.lax.broadcasted_iota(jnp.int32, (BLOCK, G), 0) + i * BLOCK
    one_hot = (row >= lo_ref[...]) & (row < hi_ref[...])

    masked = jnp.where(one_hot, jnp.broadcast_to(score, (BLOCK, G)), NEG)
    bm = jnp.max(masked, axis=0, keepdims=True)
    m_old = m_ref[...]
    m_new = jnp.maximum(m_old, bm)
    scale = jnp.exp(m_old - m_new)

    p = jnp.exp(masked - m_new).astype(jnp.bfloat16)

    d_ref[...] = d_ref[...] * scale + jax.lax.dot_general(
        ones_ref[...], p, (((1,), (0,)), ((), ())),
        preferred_element_type=jnp.float32,
    )
    s_ref[...] = s_ref[...] * scale + jax.lax.dot_general(
        xb.astype(jnp.bfloat16), p, (((0,), (0,)), ((), ())),
        preferred_element_type=jnp.float32,
    )
    m_ref[...] = m_new

    @pl.when(i == NB - 1)
    def _fini():
        d = d_ref[...]
        d = jnp.where(d == 0.0, 1.0, d)
        out_ref[...] = (s_ref[...] / d).T


@jax.jit
def kernel(x, graph_ptr, W, b, query):
    bounds = jnp.searchsorted(graph_ptr, jnp.arange(G + 1, dtype=jnp.int32))
    lo = bounds[:G].astype(jnp.int32).reshape(1, G)
    hi = bounds[1:].astype(jnp.int32).reshape(1, G)
    b2 = b.reshape(1, D)
    q2 = query.reshape(D, 1)
    ones = jnp.ones((1, BLOCK), jnp.bfloat16)
    return pl.pallas_call(
        _body,
        grid=(NB,),
        in_specs=[
            pl.BlockSpec((BLOCK, D), lambda i: (i, 0)),
            pl.BlockSpec((D, D), lambda i: (0, 0)),
            pl.BlockSpec((1, D), lambda i: (0, 0)),
            pl.BlockSpec((D, 1), lambda i: (0, 0)),
            pl.BlockSpec((1, BLOCK), lambda i: (0, 0)),
            pl.BlockSpec((1, G), lambda i: (0, 0)),
            pl.BlockSpec((1, G), lambda i: (0, 0)),
        ],
        out_specs=pl.BlockSpec((G, D), lambda i: (0, 0)),
        out_shape=jax.ShapeDtypeStruct((G, D), jnp.float32),
        scratch_shapes=[
            pltpu.VMEM((1, G), jnp.float32),
            pltpu.VMEM((1, G), jnp.float32),
            pltpu.VMEM((D, G), jnp.float32),
        ],
    )(x, W, b2, q2, ones, lo, hi)


# speedup vs baseline: 1.6691x; 1.6691x over previous
import jax
import jax.numpy as jnp
from jax.experimental import pallas as pl
from jax.experimental.pallas import tpu as pltpu

N = 100000
D = 128
G = 256
BLOCK = 10000
NB = N // BLOCK

NEG = -1e30


def _body(x_ref, ids_ref, w_ref, b_ref, q_ref, ones_ref,
          out_ref, m_ref, d_ref, s_ref):
    i = pl.program_id(0)

    @pl.when(i == 0)
    def _init():
        m_ref[...] = jnp.full((1, G), NEG, jnp.float32)
        d_ref[...] = jnp.zeros((1, G), jnp.float32)
        s_ref[...] = jnp.zeros((D, G), jnp.float32)

    xb = x_ref[...]
    g = jnp.tanh(
        jax.lax.dot_general(
            xb, w_ref[...], (((1,), (1,)), ((), ())),
            preferred_element_type=jnp.float32,
        )
        + b_ref[...]
    )
    score = jax.lax.dot_general(
        g, q_ref[...], (((1,), (0,)), ((), ())),
        preferred_element_type=jnp.float32,
    )

    ids = ids_ref[0].reshape(1, BLOCK).T
    one_hot = ids == jax.lax.broadcasted_iota(jnp.int32, (BLOCK, G), 1)

    masked = jnp.where(one_hot, jnp.broadcast_to(score, (BLOCK, G)), NEG)
    bm = jnp.max(masked, axis=0, keepdims=True)
    m_old = m_ref[...]
    m_new = jnp.maximum(m_old, bm)
    scale = jnp.exp(m_old - m_new)

    p = jnp.exp(masked - m_new).astype(jnp.bfloat16)

    d_ref[...] = d_ref[...] * scale + jax.lax.dot_general(
        ones_ref[...], p, (((1,), (0,)), ((), ())),
        preferred_element_type=jnp.float32,
    )
    s_ref[...] = s_ref[...] * scale + jax.lax.dot_general(
        xb.astype(jnp.bfloat16), p, (((0,), (0,)), ((), ())),
        preferred_element_type=jnp.float32,
    )
    m_ref[...] = m_new

    @pl.when(i == NB - 1)
    def _fini():
        d = d_ref[...]
        d = jnp.where(d == 0.0, 1.0, d)
        out_ref[...] = (s_ref[...] / d).T


@jax.jit
def kernel(x, graph_ptr, W, b, query):
    ids = graph_ptr.reshape(NB, 1, BLOCK)
    b2 = b.reshape(1, D)
    q2 = query.reshape(D, 1)
    ones = jnp.ones((1, BLOCK), jnp.bfloat16)
    return pl.pallas_call(
        _body,
        grid=(NB,),
        in_specs=[
            pl.BlockSpec((BLOCK, D), lambda i: (i, 0)),
            pl.BlockSpec((1, 1, BLOCK), lambda i: (i, 0, 0)),
            pl.BlockSpec((D, D), lambda i: (0, 0)),
            pl.BlockSpec((1, D), lambda i: (0, 0)),
            pl.BlockSpec((D, 1), lambda i: (0, 0)),
            pl.BlockSpec((1, BLOCK), lambda i: (0, 0)),
        ],
        out_specs=pl.BlockSpec((G, D), lambda i: (0, 0)),
        out_shape=jax.ShapeDtypeStruct((G, D), jnp.float32),
        scratch_shapes=[
            pltpu.VMEM((1, G), jnp.float32),
            pltpu.VMEM((1, G), jnp.float32),
            pltpu.VMEM((D, G), jnp.float32),
        ],
    )(x, ids, W, b2, q2, ones)


# speedup vs baseline: 2.3185x; 1.3891x over previous
import jax
import jax.numpy as jnp
from jax.experimental import pallas as pl
from jax.experimental.pallas import tpu as pltpu

N = 100000
D = 128
G = 256
BLOCK = 10000
NB = N // BLOCK

NEG = -1e30


def _body(x_ref, ids_ref, w_ref, b_ref, q_ref, out_ref, m_ref, d_ref, s_ref):
    i = pl.program_id(0)

    @pl.when(i == 0)
    def _init():
        m_ref[...] = jnp.full((G, 1), NEG, jnp.float32)
        d_ref[...] = jnp.zeros((G, 1), jnp.float32)
        s_ref[...] = jnp.zeros((G, D), jnp.float32)

    xb = x_ref[...]
    g = jnp.tanh(
        jax.lax.dot_general(
            xb, w_ref[...], (((1,), (1,)), ((), ())),
            preferred_element_type=jnp.float32,
        )
        + b_ref[...]
    )
    score = jax.lax.dot_general(
        q_ref[...], g, (((1,), (1,)), ((), ())),
        preferred_element_type=jnp.float32,
    )
    mb = jnp.max(score, axis=1, keepdims=True)
    es = jnp.exp(score - mb)

    ids = ids_ref[0]
    one_hot = jax.lax.broadcasted_iota(jnp.int32, (G, BLOCK), 0) == \
        jnp.broadcast_to(ids, (G, BLOCK))
    ef = jnp.where(one_hot, jnp.broadcast_to(es, (G, BLOCK)), 0.0)
    ebf = ef.astype(jnp.bfloat16)

    bmx = jnp.max(ef, axis=1, keepdims=True)
    bm = mb + jnp.log(bmx)
    m_old = m_ref[...]
    m_new = jnp.maximum(m_old, bm)
    scale_old = jnp.exp(m_old - m_new)
    sb = jnp.exp(0.5 * jnp.minimum(mb - m_new, 104.0))

    es_sum = jnp.sum(ef, axis=1, keepdims=True)
    smat = jax.lax.dot_general(
        ebf, xb.astype(jnp.bfloat16), (((1,), (0,)), ((), ())),
        preferred_element_type=jnp.float32,
    )

    d_ref[...] = d_ref[...] * scale_old + es_sum * sb * sb
    s_ref[...] = s_ref[...] * scale_old + smat * sb * sb
    m_ref[...] = m_new

    @pl.when(i == NB - 1)
    def _fini():
        d = d_ref[...]
        d = jnp.where(d == 0.0, 1.0, d)
        out_ref[...] = s_ref[...] / d


@jax.jit
def kernel(x, graph_ptr, W, b, query):
    ids = graph_ptr.reshape(NB, 1, BLOCK)
    b2 = b.reshape(1, D)
    q2 = query.reshape(1, D)
    return pl.pallas_call(
        _body,
        grid=(NB,),
        in_specs=[
            pl.BlockSpec((BLOCK, D), lambda i: (i, 0)),
            pl.BlockSpec((1, 1, BLOCK), lambda i: (i, 0, 0)),
            pl.BlockSpec((D, D), lambda i: (0, 0)),
            pl.BlockSpec((1, D), lambda i: (0, 0)),
            pl.BlockSpec((1, D), lambda i: (0, 0)),
        ],
        out_specs=pl.BlockSpec((G, D), lambda i: (0, 0)),
        out_shape=jax.ShapeDtypeStruct((G, D), jnp.float32),
        scratch_shapes=[
            pltpu.VMEM((G, 1), jnp.float32),
            pltpu.VMEM((G, 1), jnp.float32),
            pltpu.VMEM((G, D), jnp.float32),
        ],
    )(x, ids, W, b2, q2)
